# baseline (speedup 1.0000x reference)
import functools

import jax
import jax.numpy as jnp
from jax import lax
from jax.experimental import pallas as pl
from jax.experimental.pallas import tpu as pltpu
from jax.experimental.pallas import tpu_sc as plsc

B, T, C, H, M = 2, 2048, 1024, 16, 8192
DH = C // H
TOPK = 3
BT = B * T

NC, NS = 2, 16
NW = NC * NS

F32 = jnp.float32
BF16 = jnp.bfloat16


_TB_A = 256


_NTT = T // _TB_A


def _qkv_body(x_ref, w_ref, q_ref, qh_ref, kh_ref, vh_ref, kv_ref):
    xx = x_ref[...].astype(BF16)
    qkv = jnp.dot(xx, w_ref[...].astype(BF16), preferred_element_type=F32)
    q_ref[...] = qkv[:, :C]
    kk = qkv[:, C:2 * C]
    vv = qkv[:, 2 * C:]
    kv_ref[:, 0, :] = kk
    kv_ref[:, 1, :] = vv
    for h in range(H):
        qh_ref[0, h] = qkv[:, h * DH:(h + 1) * DH]
        kh_ref[0, h] = kk[:, h * DH:(h + 1) * DH]
        vh_ref[0, h] = vv[:, h * DH:(h + 1) * DH]


def _qkv_call(x2, W_attn):
    grid = (BT // _TB_A,)
    hspec = pl.BlockSpec((1, H, _TB_A, DH),
                         lambda i: (i // _NTT, 0, i % _NTT, 0))
    return pl.pallas_call(
        _qkv_body,
        grid=grid,
        in_specs=[
            pl.BlockSpec((_TB_A, C), lambda i: (i, 0)),
            pl.BlockSpec((C, 3 * C), lambda i: (0, 0)),
        ],
        out_specs=[
            pl.BlockSpec((_TB_A, C), lambda i: (i, 0)),
            hspec, hspec, hspec,
            pl.BlockSpec((_TB_A, 2, C), lambda i: (i, 0, 0)),
        ],
        out_shape=[
            jax.ShapeDtypeStruct((BT, C), F32),
            jax.ShapeDtypeStruct((B, H, T, DH), F32),
            jax.ShapeDtypeStruct((B, H, T, DH), F32),
            jax.ShapeDtypeStruct((B, H, T, DH), F32),
            jax.ShapeDtypeStruct((BT, 2, C), F32),
        ],
    )(x2, W_attn)


_TB_B = 256


def _sdpa_body(q_ref, k_ref, v_ref, y_ref):
    tt = pl.program_id(2)
    q = q_ref[0, 0].astype(BF16)
    k = k_ref[0, 0].astype(BF16)
    v = v_ref[0, 0].astype(BF16)
    s = lax.dot_general(q, k, (((1,), (1,)), ((), ())),
                        preferred_element_type=F32)
    s = s * F32(1.0 / float(DH) ** 0.5)
    row = tt * _TB_B + lax.broadcasted_iota(jnp.int32, (_TB_B, T), 0)
    col = lax.broadcasted_iota(jnp.int32, (_TB_B, T), 1)
    s = jnp.where(col <= row, s, F32(-1e30))
    m = jnp.max(s, axis=1, keepdims=True)
    p = jnp.exp(s - m)
    l = jnp.sum(p, axis=1, keepdims=True)
    y = jnp.dot(p.astype(BF16), v, preferred_element_type=F32)
    y_ref[0, 0] = y / l


def _sdpa_call(qh, kh, vh):
    grid = (B, H, T // _TB_B)
    return pl.pallas_call(
        _sdpa_body,
        grid=grid,
        in_specs=[
            pl.BlockSpec((1, 1, _TB_B, DH), lambda b, h, tt: (b, h, tt, 0)),
            pl.BlockSpec((1, 1, T, DH), lambda b, h, tt: (b, h, 0, 0)),
            pl.BlockSpec((1, 1, T, DH), lambda b, h, tt: (b, h, 0, 0)),
        ],
        out_specs=pl.BlockSpec((1, 1, _TB_B, DH),
                               lambda b, h, tt: (b, h, tt, 0)),
        out_shape=jax.ShapeDtypeStruct((B, H, T, DH), F32),
    )(qh, kh, vh)


_TB_C = 256
_MB_C = 2048
_NM = M // _MB_C
def _knn_body(q_ref, mkv_ref, idx_ref):
    b = pl.program_id(0)
    mm = pl.program_id(1)
    tt = pl.program_id(2)
    rows = pl.ds(tt * _TB_C, _TB_C)
    q = q_ref[0].astype(BF16)
    mk = mkv_ref[0, :, 0, :].astype(BF16)
    s = lax.dot_general(q, mk, (((1,), (1,)), ((), ())),
                        preferred_element_type=F32)
    gcol = mm * _MB_C + lax.broadcasted_iota(jnp.int32, (_TB_C, _MB_C), 1)

    tvals, tidxs = [], []
    for _r in range(TOPK):
        vmax = jnp.max(s, axis=1, keepdims=True)
        imax = jnp.min(jnp.where(s == vmax, gcol, M), axis=1, keepdims=True)
        tvals.append(vmax)
        tidxs.append(imax)
        s = jnp.where(gcol == imax, F32(-jnp.inf), s)
    newv = jnp.concatenate(tvals, axis=1)
    newi = jnp.concatenate(tidxs, axis=1)

    prevv = lax.bitcast_convert_type(idx_ref[0, rows, 0:3], F32)
    previ = idx_ref[0, rows, 3:6]
    prevv = jnp.where(mm == 0, F32(-jnp.inf), prevv)
    previ = jnp.where(mm == 0, jnp.int32(M), previ)
    allv = jnp.concatenate([newv, prevv], axis=1)
    alli = jnp.concatenate([newi, previ], axis=1)
    bvals, bidxs = [], []
    for _r in range(TOPK):
        vmax = jnp.max(allv, axis=1, keepdims=True)
        imin = jnp.min(jnp.where(allv == vmax, alli, M), axis=1,
                       keepdims=True)
        bvals.append(vmax)
        bidxs.append(imin)
        allv = jnp.where((allv == vmax) & (alli == imin), F32(-jnp.inf),
                         allv)
    bv = jnp.concatenate(bvals, axis=1)
    bi = jnp.concatenate(bidxs, axis=1)

    @pl.when(mm < _NM - 1)
    def _():
        idx_ref[0, rows, 0:3] = lax.bitcast_convert_type(bv, jnp.int32)
        idx_ref[0, rows, 3:6] = bi

    @pl.when(mm == _NM - 1)
    def _():
        gidx = bi + b * M
        idx_ref[0, rows, :] = jnp.concatenate([gidx, gidx, gidx[:, 0:2]],
                                              axis=1)


def _knn_call(q3, mem_kv):
    grid = (B, _NM, T // _TB_C)
    return pl.pallas_call(
        _knn_body,
        grid=grid,
        in_specs=[
            pl.BlockSpec((1, _TB_C, C), lambda b, mm, tt: (b, tt, 0)),
            pl.BlockSpec((1, _MB_C, 2, C), lambda b, mm, tt: (b, mm, 0, 0)),
        ],
        out_specs=pl.BlockSpec((1, T, 8), lambda b, mm, tt: (b, 0, 0)),
        out_shape=jax.ShapeDtypeStruct((B, T, 8), jnp.int32),
    )(q3, mem_kv)


_NIDX = B * T * TOPK
_ROWS_W = _NIDX // NW
_CHUNK = 24
_NCHUNK = _ROWS_W // _CHUNK


def _gather_body(table_hbm, idx_hbm, out_hbm, idx_v, rows_v, sem):
    wid = lax.axis_index("s") * NC + lax.axis_index("c")
    base = wid * _ROWS_W
    pltpu.sync_copy(idx_hbm.at[pl.ds(base, _ROWS_W)], idx_v)
    for ci in range(_NCHUNK):
        pltpu.async_copy(
            table_hbm.at[idx_v.at[pl.ds(ci * _CHUNK, _CHUNK)]],
            rows_v, sem).wait()
        pltpu.sync_copy(rows_v, out_hbm.at[pl.ds(base + ci * _CHUNK, _CHUNK)])


def _gather_call(table, idx_flat):
    mesh = plsc.VectorSubcoreMesh(core_axis_name="c", subcore_axis_name="s")
    k = pl.kernel(
        _gather_body,
        out_type=jax.ShapeDtypeStruct((_NIDX, 2, C), F32),
        mesh=mesh,
        scratch_types=[
            pltpu.VMEM((_ROWS_W,), jnp.int32),
            pltpu.VMEM((_CHUNK, 2, C), F32),
            pltpu.SemaphoreType.DMA,
        ],
    )
    return k(table, idx_flat)


_TB_E = 256


def _combine_body(q_ref, y_ref, g_ref, wp_ref, gate_ref, out_ref):
    q = q_ref[0]
    y = jnp.concatenate([y_ref[0, h] for h in range(H)], axis=1)

    r16 = lax.broadcasted_iota(jnp.int32, (C, H), 0) // DH
    c16 = lax.broadcasted_iota(jnp.int32, (C, H), 1)
    E16 = (r16 == c16).astype(BF16)
    r64 = lax.broadcasted_iota(jnp.int32, (H, C), 0)
    c64 = lax.broadcasted_iota(jnp.int32, (H, C), 1) // DH
    E64 = (r64 == c64).astype(BF16)

    logits = []
    for kk in range(TOPK):
        gk = g_ref[0, :, kk, 0, :]
        logits.append(jnp.dot((q * gk).astype(BF16), E16,
                              preferred_element_type=F32)
                      * F32(0.125))
    mx = jnp.maximum(jnp.maximum(logits[0], logits[1]), logits[2])
    ws = [jnp.exp(lg - mx) for lg in logits]
    denom = ws[0] + ws[1] + ws[2]
    acc = jnp.zeros((_TB_E, C), F32)
    for kk in range(TOPK):
        gv = g_ref[0, :, kk, 1, :]
        wexp = jnp.dot(ws[kk].astype(BF16), E64, preferred_element_type=F32)
        acc = acc + wexp * gv
    den_exp = jnp.dot(denom.astype(BF16), E64, preferred_element_type=F32)
    mem_qkv = acc / den_exp

    gate = gate_ref[0:1, :]
    combined = mem_qkv * gate + y * (1.0 - gate)
    out_ref[0] = jnp.dot(combined.astype(BF16), wp_ref[...].astype(BF16),
                         preferred_element_type=F32)


def _combine_call(q3, yh, g4, W_proj, gate_row):
    grid = (B, T // _TB_E)
    return pl.pallas_call(
        _combine_body,
        grid=grid,
        in_specs=[
            pl.BlockSpec((1, _TB_E, C), lambda b, tt: (b, tt, 0)),
            pl.BlockSpec((1, H, _TB_E, DH), lambda b, tt: (b, 0, tt, 0)),
            pl.BlockSpec((1, _TB_E, TOPK, 2, C),
                         lambda b, tt: (b, tt, 0, 0, 0)),
            pl.BlockSpec((C, C), lambda b, tt: (0, 0)),
            pl.BlockSpec((8, C), lambda b, tt: (0, 0)),
        ],
        out_specs=pl.BlockSpec((1, _TB_E, C), lambda b, tt: (b, tt, 0)),
        out_shape=jax.ShapeDtypeStruct((B, T, C), F32),
    )(q3, yh, g4, W_proj, gate_row)


def kernel(x, mem_kv, W_attn, W_proj, gate_bias):
    x2 = x.reshape(BT, C)
    q, qh, kh, vh, kvmem = _qkv_call(x2, W_attn)

    q3 = q.reshape(B, T, C)
    yh = _sdpa_call(qh, kh, vh)

    idx8 = _knn_call(q3, mem_kv)
    idx_flat = idx8[:, :, :TOPK].reshape(_NIDX)

    table = mem_kv.reshape(B * M, 2, C)
    g = _gather_call(table, idx_flat)
    g4 = g.reshape(B, T, TOPK, 2, C)

    gate_vec = jnp.repeat(gate_bias.reshape(H), DH)
    gate_row = jnp.broadcast_to(gate_vec, (8, C))

    out = _combine_call(q3, yh, g4, W_proj, gate_row)
    kv_memories = kvmem.reshape(B, T, 2, C)
    return out, kv_memories


# speedup vs baseline: 1.3798x; 1.3798x over previous
import functools

import jax
import jax.numpy as jnp
from jax import lax
from jax.experimental import pallas as pl
from jax.experimental.pallas import tpu as pltpu
from jax.experimental.pallas import tpu_sc as plsc

B, T, C, H, M = 2, 2048, 1024, 16, 8192
DH = C // H
TOPK = 3
BT = B * T

NC, NS = 2, 16
NW = NC * NS

F32 = jnp.float32
BF16 = jnp.bfloat16


_TB_A = 256


_NTT = T // _TB_A


def _qkv_body(x_ref, w_ref, q_ref, qh_ref, kh_ref, vh_ref, kv_ref):
    xx = x_ref[...].astype(BF16)
    qkv = jnp.dot(xx, w_ref[...].astype(BF16), preferred_element_type=F32)
    q_ref[...] = qkv[:, :C]
    kk = qkv[:, C:2 * C]
    vv = qkv[:, 2 * C:]
    kv_ref[:, 0, :] = kk
    kv_ref[:, 1, :] = vv
    for h in range(H):
        qh_ref[0, h] = qkv[:, h * DH:(h + 1) * DH]
        kh_ref[0, h] = kk[:, h * DH:(h + 1) * DH]
        vh_ref[0, h] = vv[:, h * DH:(h + 1) * DH]


def _qkv_call(x2, W_attn):
    grid = (BT // _TB_A,)
    hspec = pl.BlockSpec((1, H, _TB_A, DH),
                         lambda i: (i // _NTT, 0, i % _NTT, 0))
    return pl.pallas_call(
        _qkv_body,
        grid=grid,
        in_specs=[
            pl.BlockSpec((_TB_A, C), lambda i: (i, 0)),
            pl.BlockSpec((C, 3 * C), lambda i: (0, 0)),
        ],
        out_specs=[
            pl.BlockSpec((_TB_A, C), lambda i: (i, 0)),
            hspec, hspec, hspec,
            pl.BlockSpec((_TB_A, 2, C), lambda i: (i, 0, 0)),
        ],
        out_shape=[
            jax.ShapeDtypeStruct((BT, C), F32),
            jax.ShapeDtypeStruct((B, H, T, DH), F32),
            jax.ShapeDtypeStruct((B, H, T, DH), F32),
            jax.ShapeDtypeStruct((B, H, T, DH), F32),
            jax.ShapeDtypeStruct((BT, 2, C), F32),
        ],
    )(x2, W_attn)


_TB_B = 256


def _sdpa_body(q_ref, k_ref, v_ref, y_ref):
    tt = pl.program_id(2)
    q = q_ref[0, 0].astype(BF16)
    k = k_ref[0, 0].astype(BF16)
    v = v_ref[0, 0].astype(BF16)
    s = lax.dot_general(q, k, (((1,), (1,)), ((), ())),
                        preferred_element_type=F32)
    s = s * F32(1.0 / float(DH) ** 0.5)
    row = tt * _TB_B + lax.broadcasted_iota(jnp.int32, (_TB_B, T), 0)
    col = lax.broadcasted_iota(jnp.int32, (_TB_B, T), 1)
    s = jnp.where(col <= row, s, F32(-1e30))
    m = jnp.max(s, axis=1, keepdims=True)
    p = jnp.exp(s - m)
    l = jnp.sum(p, axis=1, keepdims=True)
    y = jnp.dot(p.astype(BF16), v, preferred_element_type=F32)
    y_ref[0, 0] = y / l


def _sdpa_call(qh, kh, vh):
    grid = (B, H, T // _TB_B)
    return pl.pallas_call(
        _sdpa_body,
        grid=grid,
        in_specs=[
            pl.BlockSpec((1, 1, _TB_B, DH), lambda b, h, tt: (b, h, tt, 0)),
            pl.BlockSpec((1, 1, T, DH), lambda b, h, tt: (b, h, 0, 0)),
            pl.BlockSpec((1, 1, T, DH), lambda b, h, tt: (b, h, 0, 0)),
        ],
        out_specs=pl.BlockSpec((1, 1, _TB_B, DH),
                               lambda b, h, tt: (b, h, tt, 0)),
        out_shape=jax.ShapeDtypeStruct((B, H, T, DH), F32),
    )(qh, kh, vh)


_TB_C = 256
_MB_C = 2048
_NM = M // _MB_C
def _knn_body(q_ref, mkv_ref, idx_ref):
    b = pl.program_id(0)
    mm = pl.program_id(1)
    tt = pl.program_id(2)
    rows = pl.ds(tt * _TB_C, _TB_C)
    q = q_ref[0].astype(BF16)
    mk = mkv_ref[0, :, :C].astype(BF16)
    s = lax.dot_general(q, mk, (((1,), (1,)), ((), ())),
                        preferred_element_type=F32)
    gcol = mm * _MB_C + lax.broadcasted_iota(jnp.int32, (_TB_C, _MB_C), 1)

    tvals, tidxs = [], []
    for _r in range(TOPK):
        vmax = jnp.max(s, axis=1, keepdims=True)
        imax = jnp.min(jnp.where(s == vmax, gcol, M), axis=1, keepdims=True)
        tvals.append(vmax)
        tidxs.append(imax)
        s = jnp.where(gcol == imax, F32(-jnp.inf), s)
    newv = jnp.concatenate(tvals, axis=1)
    newi = jnp.concatenate(tidxs, axis=1)

    prevv = lax.bitcast_convert_type(idx_ref[0, rows, 0:3], F32)
    previ = idx_ref[0, rows, 3:6]
    prevv = jnp.where(mm == 0, F32(-jnp.inf), prevv)
    previ = jnp.where(mm == 0, jnp.int32(M), previ)
    allv = jnp.concatenate([newv, prevv], axis=1)
    alli = jnp.concatenate([newi, previ], axis=1)
    bvals, bidxs = [], []
    for _r in range(TOPK):
        vmax = jnp.max(allv, axis=1, keepdims=True)
        imin = jnp.min(jnp.where(allv == vmax, alli, M), axis=1,
                       keepdims=True)
        bvals.append(vmax)
        bidxs.append(imin)
        allv = jnp.where((allv == vmax) & (alli == imin), F32(-jnp.inf),
                         allv)
    bv = jnp.concatenate(bvals, axis=1)
    bi = jnp.concatenate(bidxs, axis=1)

    @pl.when(mm < _NM - 1)
    def _():
        idx_ref[0, rows, 0:3] = lax.bitcast_convert_type(bv, jnp.int32)
        idx_ref[0, rows, 3:6] = bi

    @pl.when(mm == _NM - 1)
    def _():
        gidx = bi + b * M
        idx_ref[0, rows, :] = jnp.concatenate([gidx, gidx, gidx[:, 0:2]],
                                              axis=1)


def _knn_call(q3, mem_kv):
    grid = (B, _NM, T // _TB_C)
    return pl.pallas_call(
        _knn_body,
        grid=grid,
        in_specs=[
            pl.BlockSpec((1, _TB_C, C), lambda b, mm, tt: (b, tt, 0)),
            pl.BlockSpec((1, _MB_C, 2 * C), lambda b, mm, tt: (b, mm, 0)),
        ],
        out_specs=pl.BlockSpec((1, T, 8), lambda b, mm, tt: (b, 0, 0)),
        out_shape=jax.ShapeDtypeStruct((B, T, 8), jnp.int32),
    )(q3, mem_kv)


_NIDX = B * T * TOPK
_ROWS_W = _NIDX // NW
_CHUNK = 24
_NCHUNK = _ROWS_W // _CHUNK


def _gather_body(table_hbm, idx_hbm, out_hbm, idx_v, rows_v, sem):
    wid = lax.axis_index("s") * NC + lax.axis_index("c")
    base = wid * _ROWS_W
    pltpu.sync_copy(idx_hbm.at[pl.ds(base, _ROWS_W)], idx_v)
    for ci in range(_NCHUNK):
        pltpu.async_copy(
            table_hbm.at[idx_v.at[pl.ds(ci * _CHUNK, _CHUNK)]],
            rows_v, sem).wait()
        pltpu.sync_copy(rows_v, out_hbm.at[pl.ds(base + ci * _CHUNK, _CHUNK)])


def _gather_call(table, idx_flat):
    mesh = plsc.VectorSubcoreMesh(core_axis_name="c", subcore_axis_name="s")
    k = pl.kernel(
        _gather_body,
        out_type=jax.ShapeDtypeStruct((_NIDX, 2, C), F32),
        mesh=mesh,
        scratch_types=[
            pltpu.VMEM((_ROWS_W,), jnp.int32),
            pltpu.VMEM((_CHUNK, 2, C), F32),
            pltpu.SemaphoreType.DMA,
        ],
    )
    return k(table, idx_flat)


_TB_E = 256


def _combine_body(q_ref, y_ref, g_ref, wp_ref, gate_ref, out_ref):
    q = q_ref[0]
    y = jnp.concatenate([y_ref[0, h] for h in range(H)], axis=1)

    r16 = lax.broadcasted_iota(jnp.int32, (C, H), 0) // DH
    c16 = lax.broadcasted_iota(jnp.int32, (C, H), 1)
    E16 = (r16 == c16).astype(BF16)
    r64 = lax.broadcasted_iota(jnp.int32, (H, C), 0)
    c64 = lax.broadcasted_iota(jnp.int32, (H, C), 1) // DH
    E64 = (r64 == c64).astype(BF16)

    logits = []
    for kk in range(TOPK):
        gk = g_ref[0, :, kk, 0, :]
        logits.append(jnp.dot((q * gk).astype(BF16), E16,
                              preferred_element_type=F32)
                      * F32(0.125))
    mx = jnp.maximum(jnp.maximum(logits[0], logits[1]), logits[2])
    ws = [jnp.exp(lg - mx) for lg in logits]
    denom = ws[0] + ws[1] + ws[2]
    acc = jnp.zeros((_TB_E, C), F32)
    for kk in range(TOPK):
        gv = g_ref[0, :, kk, 1, :]
        wexp = jnp.dot(ws[kk].astype(BF16), E64, preferred_element_type=F32)
        acc = acc + wexp * gv
    den_exp = jnp.dot(denom.astype(BF16), E64, preferred_element_type=F32)
    mem_qkv = acc / den_exp

    gate = gate_ref[0:1, :]
    combined = mem_qkv * gate + y * (1.0 - gate)
    out_ref[0] = jnp.dot(combined.astype(BF16), wp_ref[...].astype(BF16),
                         preferred_element_type=F32)


def _combine_call(q3, yh, g4, W_proj, gate_row):
    grid = (B, T // _TB_E)
    return pl.pallas_call(
        _combine_body,
        grid=grid,
        in_specs=[
            pl.BlockSpec((1, _TB_E, C), lambda b, tt: (b, tt, 0)),
            pl.BlockSpec((1, H, _TB_E, DH), lambda b, tt: (b, 0, tt, 0)),
            pl.BlockSpec((1, _TB_E, TOPK, 2, C),
                         lambda b, tt: (b, tt, 0, 0, 0)),
            pl.BlockSpec((C, C), lambda b, tt: (0, 0)),
            pl.BlockSpec((8, C), lambda b, tt: (0, 0)),
        ],
        out_specs=pl.BlockSpec((1, _TB_E, C), lambda b, tt: (b, tt, 0)),
        out_shape=jax.ShapeDtypeStruct((B, T, C), F32),
    )(q3, yh, g4, W_proj, gate_row)


def kernel(x, mem_kv, W_attn, W_proj, gate_bias):
    x2 = x.reshape(BT, C)
    q, qh, kh, vh, kvmem = _qkv_call(x2, W_attn)

    q3 = q.reshape(B, T, C)
    yh = _sdpa_call(qh, kh, vh)

    idx8 = _knn_call(q3, mem_kv.reshape(B, M, 2 * C))
    idx_flat = idx8[:, :, :TOPK].reshape(_NIDX)

    table = mem_kv.reshape(B * M, 2, C)
    g = _gather_call(table, idx_flat)
    g4 = g.reshape(B, T, TOPK, 2, C)

    gate_vec = jnp.repeat(gate_bias.reshape(H), DH)
    gate_row = jnp.broadcast_to(gate_vec, (8, C))

    out = _combine_call(q3, yh, g4, W_proj, gate_row)
    kv_memories = kvmem.reshape(B, T, 2, C)
    return out, kv_memories


# speedup vs baseline: 1.5136x; 1.0970x over previous
import functools

import jax
import jax.numpy as jnp
from jax import lax
from jax.experimental import pallas as pl
from jax.experimental.pallas import tpu as pltpu
from jax.experimental.pallas import tpu_sc as plsc

B, T, C, H, M = 2, 2048, 1024, 16, 8192
DH = C // H
TOPK = 3
BT = B * T

NC, NS = 2, 16
NW = NC * NS

F32 = jnp.float32
BF16 = jnp.bfloat16


_TB_A = 256


_NTT = T // _TB_A


def _qkv_body(x_ref, w_ref, q_ref, qh_ref, kh_ref, vh_ref, kv_ref):
    xx = x_ref[...].astype(BF16)
    qkv = jnp.dot(xx, w_ref[...].astype(BF16), preferred_element_type=F32)
    q_ref[...] = qkv[:, :C]
    kk = qkv[:, C:2 * C]
    vv = qkv[:, 2 * C:]
    kv_ref[:, 0, :] = kk
    kv_ref[:, 1, :] = vv
    for h in range(H):
        qh_ref[0, h] = qkv[:, h * DH:(h + 1) * DH]
        kh_ref[0, h] = kk[:, h * DH:(h + 1) * DH]
        vh_ref[0, h] = vv[:, h * DH:(h + 1) * DH]


def _qkv_call(x2, W_attn):
    grid = (BT // _TB_A,)
    hspec = pl.BlockSpec((1, H, _TB_A, DH),
                         lambda i: (i // _NTT, 0, i % _NTT, 0))
    return pl.pallas_call(
        _qkv_body,
        grid=grid,
        in_specs=[
            pl.BlockSpec((_TB_A, C), lambda i: (i, 0)),
            pl.BlockSpec((C, 3 * C), lambda i: (0, 0)),
        ],
        out_specs=[
            pl.BlockSpec((_TB_A, C), lambda i: (i, 0)),
            hspec, hspec, hspec,
            pl.BlockSpec((_TB_A, 2, C), lambda i: (i, 0, 0)),
        ],
        out_shape=[
            jax.ShapeDtypeStruct((BT, C), F32),
            jax.ShapeDtypeStruct((B, H, T, DH), F32),
            jax.ShapeDtypeStruct((B, H, T, DH), F32),
            jax.ShapeDtypeStruct((B, H, T, DH), F32),
            jax.ShapeDtypeStruct((BT, 2, C), F32),
        ],
    )(x2, W_attn)


_TB_B = 256


def _sdpa_body(q_ref, k_ref, v_ref, y_ref):
    tt = pl.program_id(2)
    q = q_ref[0, 0].astype(BF16)
    k = k_ref[0, 0].astype(BF16)
    v = v_ref[0, 0].astype(BF16)
    s = lax.dot_general(q, k, (((1,), (1,)), ((), ())),
                        preferred_element_type=F32)
    s = s * F32(1.0 / float(DH) ** 0.5)
    row = tt * _TB_B + lax.broadcasted_iota(jnp.int32, (_TB_B, T), 0)
    col = lax.broadcasted_iota(jnp.int32, (_TB_B, T), 1)
    s = jnp.where(col <= row, s, F32(-1e30))
    m = jnp.max(s, axis=1, keepdims=True)
    p = jnp.exp(s - m)
    l = jnp.sum(p, axis=1, keepdims=True)
    y = jnp.dot(p.astype(BF16), v, preferred_element_type=F32)
    y_ref[0, 0] = y / l


def _sdpa_call(qh, kh, vh):
    grid = (B, H, T // _TB_B)
    return pl.pallas_call(
        _sdpa_body,
        grid=grid,
        in_specs=[
            pl.BlockSpec((1, 1, _TB_B, DH), lambda b, h, tt: (b, h, tt, 0)),
            pl.BlockSpec((1, 1, T, DH), lambda b, h, tt: (b, h, 0, 0)),
            pl.BlockSpec((1, 1, T, DH), lambda b, h, tt: (b, h, 0, 0)),
        ],
        out_specs=pl.BlockSpec((1, 1, _TB_B, DH),
                               lambda b, h, tt: (b, h, tt, 0)),
        out_shape=jax.ShapeDtypeStruct((B, H, T, DH), F32),
    )(qh, kh, vh)


_TB_C = 256
_MB_C = 2048
_NM = M // _MB_C
def _knn_body(q_ref, mkv_ref, idx_ref):
    b = pl.program_id(0)
    mm = pl.program_id(1)
    tt = pl.program_id(2)
    rows = pl.ds(tt * _TB_C, _TB_C)
    q = q_ref[0].astype(BF16)
    mk = mkv_ref[0].astype(BF16)
    s = lax.dot_general(q, mk, (((1,), (1,)), ((), ())),
                        preferred_element_type=F32)
    gcol = mm * _MB_C + lax.broadcasted_iota(jnp.int32, (_TB_C, _MB_C), 1)

    tvals, tidxs = [], []
    for _r in range(TOPK):
        vmax = jnp.max(s, axis=1, keepdims=True)
        imax = jnp.min(jnp.where(s == vmax, gcol, M), axis=1, keepdims=True)
        tvals.append(vmax)
        tidxs.append(imax)
        s = jnp.where(gcol == imax, F32(-jnp.inf), s)
    newv = jnp.concatenate(tvals, axis=1)
    newi = jnp.concatenate(tidxs, axis=1)

    prevv = lax.bitcast_convert_type(idx_ref[0, rows, 0:3], F32)
    previ = idx_ref[0, rows, 3:6]
    prevv = jnp.where(mm == 0, F32(-jnp.inf), prevv)
    previ = jnp.where(mm == 0, jnp.int32(M), previ)
    allv = jnp.concatenate([newv, prevv], axis=1)
    alli = jnp.concatenate([newi, previ], axis=1)
    bvals, bidxs = [], []
    for _r in range(TOPK):
        vmax = jnp.max(allv, axis=1, keepdims=True)
        imin = jnp.min(jnp.where(allv == vmax, alli, M), axis=1,
                       keepdims=True)
        bvals.append(vmax)
        bidxs.append(imin)
        allv = jnp.where((allv == vmax) & (alli == imin), F32(-jnp.inf),
                         allv)
    bv = jnp.concatenate(bvals, axis=1)
    bi = jnp.concatenate(bidxs, axis=1)

    @pl.when(mm < _NM - 1)
    def _():
        idx_ref[0, rows, 0:3] = lax.bitcast_convert_type(bv, jnp.int32)
        idx_ref[0, rows, 3:6] = bi

    @pl.when(mm == _NM - 1)
    def _():
        gidx = bi + b * M
        idx_ref[0, rows, :] = jnp.concatenate([gidx, gidx, gidx[:, 0:2]],
                                              axis=1)


def _knn_call(q3, mem_kv):
    grid = (B, _NM, T // _TB_C)
    return pl.pallas_call(
        _knn_body,
        grid=grid,
        in_specs=[
            pl.BlockSpec((1, _TB_C, C), lambda b, mm, tt: (b, tt, 0)),
            pl.BlockSpec((1, _MB_C, C), lambda b, mm, tt: (b, mm, 0)),
        ],
        out_specs=pl.BlockSpec((1, T, 8), lambda b, mm, tt: (b, 0, 0)),
        out_shape=jax.ShapeDtypeStruct((B, T, 8), jnp.int32),
    )(q3, mem_kv)


_NIDX = B * T * TOPK
_ROWS_W = _NIDX // NW
_CHUNK = 24
_NCHUNK = _ROWS_W // _CHUNK


def _gather_body(table_hbm, idx_hbm, out_hbm, idx_v, rows_v, sem):
    wid = lax.axis_index("s") * NC + lax.axis_index("c")
    base = wid * _ROWS_W
    pltpu.sync_copy(idx_hbm.at[pl.ds(base, _ROWS_W)], idx_v)
    for ci in range(_NCHUNK):
        pltpu.async_copy(
            table_hbm.at[idx_v.at[pl.ds(ci * _CHUNK, _CHUNK)]],
            rows_v, sem).wait()
        pltpu.sync_copy(rows_v, out_hbm.at[pl.ds(base + ci * _CHUNK, _CHUNK)])


def _gather_call(table, idx_flat):
    mesh = plsc.VectorSubcoreMesh(core_axis_name="c", subcore_axis_name="s")
    k = pl.kernel(
        _gather_body,
        out_type=jax.ShapeDtypeStruct((_NIDX, 2, C), F32),
        mesh=mesh,
        scratch_types=[
            pltpu.VMEM((_ROWS_W,), jnp.int32),
            pltpu.VMEM((_CHUNK, 2, C), F32),
            pltpu.SemaphoreType.DMA,
        ],
    )
    return k(table, idx_flat)


_TB_E = 256


def _combine_body(q_ref, y_ref, g_ref, wp_ref, gate_ref, out_ref):
    q = q_ref[0]
    y = jnp.concatenate([y_ref[0, h] for h in range(H)], axis=1)

    r16 = lax.broadcasted_iota(jnp.int32, (C, H), 0) // DH
    c16 = lax.broadcasted_iota(jnp.int32, (C, H), 1)
    E16 = (r16 == c16).astype(BF16)
    r64 = lax.broadcasted_iota(jnp.int32, (H, C), 0)
    c64 = lax.broadcasted_iota(jnp.int32, (H, C), 1) // DH
    E64 = (r64 == c64).astype(BF16)

    logits = []
    for kk in range(TOPK):
        gk = g_ref[0, :, kk, 0, :]
        logits.append(jnp.dot((q * gk).astype(BF16), E16,
                              preferred_element_type=F32)
                      * F32(0.125))
    mx = jnp.maximum(jnp.maximum(logits[0], logits[1]), logits[2])
    ws = [jnp.exp(lg - mx) for lg in logits]
    denom = ws[0] + ws[1] + ws[2]
    acc = jnp.zeros((_TB_E, C), F32)
    for kk in range(TOPK):
        gv = g_ref[0, :, kk, 1, :]
        wexp = jnp.dot(ws[kk].astype(BF16), E64, preferred_element_type=F32)
        acc = acc + wexp * gv
    den_exp = jnp.dot(denom.astype(BF16), E64, preferred_element_type=F32)
    mem_qkv = acc / den_exp

    gate = gate_ref[0:1, :]
    combined = mem_qkv * gate + y * (1.0 - gate)
    out_ref[0] = jnp.dot(combined.astype(BF16), wp_ref[...].astype(BF16),
                         preferred_element_type=F32)


def _combine_call(q3, yh, g4, W_proj, gate_row):
    grid = (B, T // _TB_E)
    return pl.pallas_call(
        _combine_body,
        grid=grid,
        in_specs=[
            pl.BlockSpec((1, _TB_E, C), lambda b, tt: (b, tt, 0)),
            pl.BlockSpec((1, H, _TB_E, DH), lambda b, tt: (b, 0, tt, 0)),
            pl.BlockSpec((1, _TB_E, TOPK, 2, C),
                         lambda b, tt: (b, tt, 0, 0, 0)),
            pl.BlockSpec((C, C), lambda b, tt: (0, 0)),
            pl.BlockSpec((8, C), lambda b, tt: (0, 0)),
        ],
        out_specs=pl.BlockSpec((1, _TB_E, C), lambda b, tt: (b, tt, 0)),
        out_shape=jax.ShapeDtypeStruct((B, T, C), F32),
    )(q3, yh, g4, W_proj, gate_row)


def kernel(x, mem_kv, W_attn, W_proj, gate_bias):
    x2 = x.reshape(BT, C)
    q, qh, kh, vh, kvmem = _qkv_call(x2, W_attn)

    q3 = q.reshape(B, T, C)
    yh = _sdpa_call(qh, kh, vh)

    idx8 = _knn_call(q3, mem_kv[:, :, 0, :])
    idx_flat = idx8[:, :, :TOPK].reshape(_NIDX)

    table = mem_kv.reshape(B * M, 2, C)
    g = _gather_call(table, idx_flat)
    g4 = g.reshape(B, T, TOPK, 2, C)

    gate_vec = jnp.repeat(gate_bias.reshape(H), DH)
    gate_row = jnp.broadcast_to(gate_vec, (8, C))

    out = _combine_call(q3, yh, g4, W_proj, gate_row)
    kv_memories = kvmem.reshape(B, T, 2, C)
    return out, kv_memories


# speedup vs baseline: 1.6493x; 1.0896x over previous
import functools

import jax
import jax.numpy as jnp
from jax import lax
from jax.experimental import pallas as pl
from jax.experimental.pallas import tpu as pltpu
from jax.experimental.pallas import tpu_sc as plsc

B, T, C, H, M = 2, 2048, 1024, 16, 8192
DH = C // H
TOPK = 3
BT = B * T

NC, NS = 2, 16
NW = NC * NS

F32 = jnp.float32
BF16 = jnp.bfloat16


_TB_A = 256


_NTT = T // _TB_A


def _qkv_body(x_ref, w_ref, q_ref, qh_ref, kh_ref, vh_ref, kv_ref):
    xx = x_ref[...].astype(BF16)
    qkv = jnp.dot(xx, w_ref[...].astype(BF16), preferred_element_type=F32)
    q_ref[...] = qkv[:, :C]
    kk = qkv[:, C:2 * C]
    vv = qkv[:, 2 * C:]
    kv_ref[:, 0, :] = kk
    kv_ref[:, 1, :] = vv
    qkv_bf = qkv.astype(BF16)
    for h in range(H):
        qh_ref[0, h] = qkv_bf[:, h * DH:(h + 1) * DH]
        kh_ref[0, h] = qkv_bf[:, C + h * DH:C + (h + 1) * DH]
        vh_ref[0, h] = qkv_bf[:, 2 * C + h * DH:2 * C + (h + 1) * DH]


def _qkv_call(x2, W_attn):
    grid = (BT // _TB_A,)
    hspec = pl.BlockSpec((1, H, _TB_A, DH),
                         lambda i: (i // _NTT, 0, i % _NTT, 0))
    return pl.pallas_call(
        _qkv_body,
        grid=grid,
        in_specs=[
            pl.BlockSpec((_TB_A, C), lambda i: (i, 0)),
            pl.BlockSpec((C, 3 * C), lambda i: (0, 0)),
        ],
        out_specs=[
            pl.BlockSpec((_TB_A, C), lambda i: (i, 0)),
            hspec, hspec, hspec,
            pl.BlockSpec((_TB_A, 2, C), lambda i: (i, 0, 0)),
        ],
        out_shape=[
            jax.ShapeDtypeStruct((BT, C), F32),
            jax.ShapeDtypeStruct((B, H, T, DH), BF16),
            jax.ShapeDtypeStruct((B, H, T, DH), BF16),
            jax.ShapeDtypeStruct((B, H, T, DH), BF16),
            jax.ShapeDtypeStruct((BT, 2, C), F32),
        ],
    )(x2, W_attn)


_TB_B = 256


def _sdpa_body(q_ref, k_ref, v_ref, y_ref):
    tt = pl.program_id(2)
    q = q_ref[0, 0]
    k = k_ref[0, 0]
    v = v_ref[0, 0]
    s = lax.dot_general(q, k, (((1,), (1,)), ((), ())),
                        preferred_element_type=F32)
    s = s * F32(1.0 / float(DH) ** 0.5)
    row = tt * _TB_B + lax.broadcasted_iota(jnp.int32, (_TB_B, T), 0)
    col = lax.broadcasted_iota(jnp.int32, (_TB_B, T), 1)
    s = jnp.where(col <= row, s, F32(-1e30))
    m = jnp.max(s, axis=1, keepdims=True)
    p = jnp.exp(s - m)
    l = jnp.sum(p, axis=1, keepdims=True)
    y = jnp.dot(p.astype(BF16), v, preferred_element_type=F32)
    y_ref[0, 0] = y / l


def _sdpa_call(qh, kh, vh):
    grid = (B, H, T // _TB_B)
    return pl.pallas_call(
        _sdpa_body,
        grid=grid,
        in_specs=[
            pl.BlockSpec((1, 1, _TB_B, DH), lambda b, h, tt: (b, h, tt, 0)),
            pl.BlockSpec((1, 1, T, DH), lambda b, h, tt: (b, h, 0, 0)),
            pl.BlockSpec((1, 1, T, DH), lambda b, h, tt: (b, h, 0, 0)),
        ],
        out_specs=pl.BlockSpec((1, 1, _TB_B, DH),
                               lambda b, h, tt: (b, h, tt, 0)),
        out_shape=jax.ShapeDtypeStruct((B, H, T, DH), F32),
    )(qh, kh, vh)


_TB_C = 256
_MB_C = 4096
_NM = M // _MB_C
def _knn_body(q_ref, mkv_ref, idx_ref):
    b = pl.program_id(0)
    mm = pl.program_id(1)
    tt = pl.program_id(2)
    rows = pl.ds(tt * _TB_C, _TB_C)
    q = q_ref[0].astype(BF16)
    mk = mkv_ref[0].astype(BF16)
    s = lax.dot_general(q, mk, (((1,), (1,)), ((), ())),
                        preferred_element_type=F32)
    gcol = mm * _MB_C + lax.broadcasted_iota(jnp.int32, (_TB_C, _MB_C), 1)

    tvals, tidxs = [], []
    for _r in range(TOPK):
        vmax = jnp.max(s, axis=1, keepdims=True)
        imax = jnp.min(jnp.where(s == vmax, gcol, M), axis=1, keepdims=True)
        tvals.append(vmax)
        tidxs.append(imax)
        s = jnp.where(gcol == imax, F32(-jnp.inf), s)
    newv = jnp.concatenate(tvals, axis=1)
    newi = jnp.concatenate(tidxs, axis=1)

    prevv = lax.bitcast_convert_type(idx_ref[0, rows, 0:3], F32)
    previ = idx_ref[0, rows, 3:6]
    prevv = jnp.where(mm == 0, F32(-jnp.inf), prevv)
    previ = jnp.where(mm == 0, jnp.int32(M), previ)
    allv = jnp.concatenate([newv, prevv], axis=1)
    alli = jnp.concatenate([newi, previ], axis=1)
    bvals, bidxs = [], []
    for _r in range(TOPK):
        vmax = jnp.max(allv, axis=1, keepdims=True)
        imin = jnp.min(jnp.where(allv == vmax, alli, M), axis=1,
                       keepdims=True)
        bvals.append(vmax)
        bidxs.append(imin)
        allv = jnp.where((allv == vmax) & (alli == imin), F32(-jnp.inf),
                         allv)
    bv = jnp.concatenate(bvals, axis=1)
    bi = jnp.concatenate(bidxs, axis=1)

    @pl.when(mm < _NM - 1)
    def _():
        idx_ref[0, rows, 0:3] = lax.bitcast_convert_type(bv, jnp.int32)
        idx_ref[0, rows, 3:6] = bi

    @pl.when(mm == _NM - 1)
    def _():
        gidx = bi + b * M
        idx_ref[0, rows, :] = jnp.concatenate([gidx, gidx, gidx[:, 0:2]],
                                              axis=1)


def _knn_call(q3, mem_kv):
    grid = (B, _NM, T // _TB_C)
    return pl.pallas_call(
        _knn_body,
        grid=grid,
        in_specs=[
            pl.BlockSpec((1, _TB_C, C), lambda b, mm, tt: (b, tt, 0)),
            pl.BlockSpec((1, _MB_C, C), lambda b, mm, tt: (b, mm, 0)),
        ],
        out_specs=pl.BlockSpec((1, T, 8), lambda b, mm, tt: (b, 0, 0)),
        out_shape=jax.ShapeDtypeStruct((B, T, 8), jnp.int32),
    )(q3, mem_kv)


_NIDX = B * T * TOPK
_ROWS_W = _NIDX // NW
_CHUNK = 24
_NCHUNK = _ROWS_W // _CHUNK


def _gather_body(table_hbm, idx_hbm, out_hbm, idx_v, rows_v, sem):
    wid = lax.axis_index("s") * NC + lax.axis_index("c")
    base = wid * _ROWS_W
    pltpu.sync_copy(idx_hbm.at[pl.ds(base, _ROWS_W)], idx_v)
    for ci in range(_NCHUNK):
        pltpu.async_copy(
            table_hbm.at[idx_v.at[pl.ds(ci * _CHUNK, _CHUNK)]],
            rows_v, sem).wait()
        pltpu.sync_copy(rows_v, out_hbm.at[pl.ds(base + ci * _CHUNK, _CHUNK)])


def _gather_call(table, idx_flat):
    mesh = plsc.VectorSubcoreMesh(core_axis_name="c", subcore_axis_name="s")
    k = pl.kernel(
        _gather_body,
        out_type=jax.ShapeDtypeStruct((_NIDX, 2, C), F32),
        mesh=mesh,
        scratch_types=[
            pltpu.VMEM((_ROWS_W,), jnp.int32),
            pltpu.VMEM((_CHUNK, 2, C), F32),
            pltpu.SemaphoreType.DMA,
        ],
    )
    return k(table, idx_flat)


_TB_E = 256


def _combine_body(q_ref, y_ref, g_ref, wp_ref, gate_ref, out_ref):
    q = q_ref[0]
    y = jnp.concatenate([y_ref[0, h] for h in range(H)], axis=1)

    r16 = lax.broadcasted_iota(jnp.int32, (C, H), 0) // DH
    c16 = lax.broadcasted_iota(jnp.int32, (C, H), 1)
    E16 = (r16 == c16).astype(BF16)
    r64 = lax.broadcasted_iota(jnp.int32, (H, C), 0)
    c64 = lax.broadcasted_iota(jnp.int32, (H, C), 1) // DH
    E64 = (r64 == c64).astype(BF16)

    logits = []
    for kk in range(TOPK):
        gk = g_ref[0, :, kk, 0, :]
        logits.append(jnp.dot((q * gk).astype(BF16), E16,
                              preferred_element_type=F32)
                      * F32(0.125))
    mx = jnp.maximum(jnp.maximum(logits[0], logits[1]), logits[2])
    ws = [jnp.exp(lg - mx) for lg in logits]
    denom = ws[0] + ws[1] + ws[2]
    acc = jnp.zeros((_TB_E, C), F32)
    for kk in range(TOPK):
        gv = g_ref[0, :, kk, 1, :]
        wexp = jnp.dot(ws[kk].astype(BF16), E64, preferred_element_type=F32)
        acc = acc + wexp * gv
    den_exp = jnp.dot(denom.astype(BF16), E64, preferred_element_type=F32)
    mem_qkv = acc / den_exp

    gate = gate_ref[0:1, :]
    combined = mem_qkv * gate + y * (1.0 - gate)
    out_ref[0] = jnp.dot(combined.astype(BF16), wp_ref[...].astype(BF16),
                         preferred_element_type=F32)


def _combine_call(q3, yh, g4, W_proj, gate_row):
    grid = (B, T // _TB_E)
    return pl.pallas_call(
        _combine_body,
        grid=grid,
        in_specs=[
            pl.BlockSpec((1, _TB_E, C), lambda b, tt: (b, tt, 0)),
            pl.BlockSpec((1, H, _TB_E, DH), lambda b, tt: (b, 0, tt, 0)),
            pl.BlockSpec((1, _TB_E, TOPK, 2, C),
                         lambda b, tt: (b, tt, 0, 0, 0)),
            pl.BlockSpec((C, C), lambda b, tt: (0, 0)),
            pl.BlockSpec((8, C), lambda b, tt: (0, 0)),
        ],
        out_specs=pl.BlockSpec((1, _TB_E, C), lambda b, tt: (b, tt, 0)),
        out_shape=jax.ShapeDtypeStruct((B, T, C), F32),
    )(q3, yh, g4, W_proj, gate_row)


def kernel(x, mem_kv, W_attn, W_proj, gate_bias):
    x2 = x.reshape(BT, C)
    q, qh, kh, vh, kvmem = _qkv_call(x2, W_attn)

    q3 = q.reshape(B, T, C)
    yh = _sdpa_call(qh, kh, vh)

    idx8 = _knn_call(q3, mem_kv[:, :, 0, :])
    idx_flat = idx8[:, :, :TOPK].reshape(_NIDX)

    table = mem_kv.reshape(B * M, 2, C)
    g = _gather_call(table, idx_flat)
    g4 = g.reshape(B, T, TOPK, 2, C)

    gate_vec = jnp.repeat(gate_bias.reshape(H), DH)
    gate_row = jnp.broadcast_to(gate_vec, (8, C))

    out = _combine_call(q3, yh, g4, W_proj, gate_row)
    kv_memories = kvmem.reshape(B, T, 2, C)
    return out, kv_memories


# speedup vs baseline: 1.6916x; 1.0257x over previous
import functools

import jax
import jax.numpy as jnp
from jax import lax
from jax.experimental import pallas as pl
from jax.experimental.pallas import tpu as pltpu
from jax.experimental.pallas import tpu_sc as plsc

B, T, C, H, M = 2, 2048, 1024, 16, 8192
DH = C // H
TOPK = 3
BT = B * T

NC, NS = 2, 16
NW = NC * NS

F32 = jnp.float32
BF16 = jnp.bfloat16


_TB_A = 256


_NTT = T // _TB_A


def _qkv_body(x_ref, w_ref, q_ref, qh_ref, kh_ref, vh_ref, kv_ref):
    xx = x_ref[...].astype(BF16)
    qkv = jnp.dot(xx, w_ref[...].astype(BF16), preferred_element_type=F32)
    kk = qkv[:, C:2 * C]
    vv = qkv[:, 2 * C:]
    kv_ref[:, 0, :] = kk
    kv_ref[:, 1, :] = vv
    qkv_bf = qkv.astype(BF16)
    q_ref[...] = qkv_bf[:, :C]
    for h in range(H):
        qh_ref[0, h] = qkv_bf[:, h * DH:(h + 1) * DH]
        kh_ref[0, h] = qkv_bf[:, C + h * DH:C + (h + 1) * DH]
        vh_ref[0, h] = qkv_bf[:, 2 * C + h * DH:2 * C + (h + 1) * DH]


def _qkv_call(x2, W_attn):
    grid = (BT // _TB_A,)
    hspec = pl.BlockSpec((1, H, _TB_A, DH),
                         lambda i: (i // _NTT, 0, i % _NTT, 0))
    return pl.pallas_call(
        _qkv_body,
        grid=grid,
        in_specs=[
            pl.BlockSpec((_TB_A, C), lambda i: (i, 0)),
            pl.BlockSpec((C, 3 * C), lambda i: (0, 0)),
        ],
        out_specs=[
            pl.BlockSpec((_TB_A, C), lambda i: (i, 0)),
            hspec, hspec, hspec,
            pl.BlockSpec((_TB_A, 2, C), lambda i: (i, 0, 0)),
        ],
        out_shape=[
            jax.ShapeDtypeStruct((BT, C), BF16),
            jax.ShapeDtypeStruct((B, H, T, DH), BF16),
            jax.ShapeDtypeStruct((B, H, T, DH), BF16),
            jax.ShapeDtypeStruct((B, H, T, DH), BF16),
            jax.ShapeDtypeStruct((BT, 2, C), F32),
        ],
    )(x2, W_attn)


_TB_B = 256


def _sdpa_body(q_ref, k_ref, v_ref, y_ref):
    tt = pl.program_id(2)
    q = q_ref[0, 0]
    k = k_ref[0, 0]
    v = v_ref[0, 0]
    s = lax.dot_general(q, k, (((1,), (1,)), ((), ())),
                        preferred_element_type=F32)
    s = s * F32(1.0 / float(DH) ** 0.5)
    row = tt * _TB_B + lax.broadcasted_iota(jnp.int32, (_TB_B, T), 0)
    col = lax.broadcasted_iota(jnp.int32, (_TB_B, T), 1)
    s = jnp.where(col <= row, s, F32(-1e30))
    m = jnp.max(s, axis=1, keepdims=True)
    p = jnp.exp(s - m)
    l = jnp.sum(p, axis=1, keepdims=True)
    y = jnp.dot(p.astype(BF16), v, preferred_element_type=F32)
    y_ref[0, 0] = y / l


def _sdpa_call(qh, kh, vh):
    grid = (B, H, T // _TB_B)
    return pl.pallas_call(
        _sdpa_body,
        grid=grid,
        in_specs=[
            pl.BlockSpec((1, 1, _TB_B, DH), lambda b, h, tt: (b, h, tt, 0)),
            pl.BlockSpec((1, 1, T, DH), lambda b, h, tt: (b, h, 0, 0)),
            pl.BlockSpec((1, 1, T, DH), lambda b, h, tt: (b, h, 0, 0)),
        ],
        out_specs=pl.BlockSpec((1, 1, _TB_B, DH),
                               lambda b, h, tt: (b, h, tt, 0)),
        out_shape=jax.ShapeDtypeStruct((B, H, T, DH), F32),
    )(qh, kh, vh)


_TB_C = 256
_MB_C = 4096
_NM = M // _MB_C
def _knn_body(q_ref, mkv_ref, idx_ref):
    b = pl.program_id(0)
    mm = pl.program_id(1)
    tt = pl.program_id(2)
    rows = pl.ds(tt * _TB_C, _TB_C)
    q = q_ref[0]
    mk = mkv_ref[0]
    s = lax.dot_general(q, mk, (((1,), (1,)), ((), ())),
                        preferred_element_type=F32)
    gcol = mm * _MB_C + lax.broadcasted_iota(jnp.int32, (_TB_C, _MB_C), 1)

    tvals, tidxs = [], []
    for _r in range(TOPK):
        vmax = jnp.max(s, axis=1, keepdims=True)
        imax = jnp.min(jnp.where(s == vmax, gcol, M), axis=1, keepdims=True)
        tvals.append(vmax)
        tidxs.append(imax)
        s = jnp.where(gcol == imax, F32(-jnp.inf), s)
    newv = jnp.concatenate(tvals, axis=1)
    newi = jnp.concatenate(tidxs, axis=1)

    prevv = lax.bitcast_convert_type(idx_ref[0, rows, 0:3], F32)
    previ = idx_ref[0, rows, 3:6]
    prevv = jnp.where(mm == 0, F32(-jnp.inf), prevv)
    previ = jnp.where(mm == 0, jnp.int32(M), previ)
    allv = jnp.concatenate([newv, prevv], axis=1)
    alli = jnp.concatenate([newi, previ], axis=1)
    bvals, bidxs = [], []
    for _r in range(TOPK):
        vmax = jnp.max(allv, axis=1, keepdims=True)
        imin = jnp.min(jnp.where(allv == vmax, alli, M), axis=1,
                       keepdims=True)
        bvals.append(vmax)
        bidxs.append(imin)
        allv = jnp.where((allv == vmax) & (alli == imin), F32(-jnp.inf),
                         allv)
    bv = jnp.concatenate(bvals, axis=1)
    bi = jnp.concatenate(bidxs, axis=1)

    @pl.when(mm < _NM - 1)
    def _():
        idx_ref[0, rows, 0:3] = lax.bitcast_convert_type(bv, jnp.int32)
        idx_ref[0, rows, 3:6] = bi

    @pl.when(mm == _NM - 1)
    def _():
        gidx = bi + b * M
        idx_ref[0, rows, :] = jnp.concatenate([gidx, gidx, gidx[:, 0:2]],
                                              axis=1)


def _knn_call(q3, mem_kv):
    grid = (B, _NM, T // _TB_C)
    return pl.pallas_call(
        _knn_body,
        grid=grid,
        in_specs=[
            pl.BlockSpec((1, _TB_C, C), lambda b, mm, tt: (b, tt, 0)),
            pl.BlockSpec((1, _MB_C, C), lambda b, mm, tt: (b, mm, 0)),
        ],
        out_specs=pl.BlockSpec((1, T, 8), lambda b, mm, tt: (b, 0, 0)),
        out_shape=jax.ShapeDtypeStruct((B, T, 8), jnp.int32),
    )(q3, mem_kv)


_NIDX = B * T * TOPK
_ROWS_W = _NIDX // NW
_CHUNK = 24
_NCHUNK = _ROWS_W // _CHUNK


def _gather_body(table_hbm, idx_hbm, out_hbm, idx_v, rows_v, sem):
    wid = lax.axis_index("s") * NC + lax.axis_index("c")
    base = wid * _ROWS_W
    pltpu.sync_copy(idx_hbm.at[pl.ds(base, _ROWS_W)], idx_v)
    for ci in range(_NCHUNK):
        pltpu.async_copy(
            table_hbm.at[idx_v.at[pl.ds(ci * _CHUNK, _CHUNK)]],
            rows_v, sem).wait()
        pltpu.sync_copy(rows_v, out_hbm.at[pl.ds(base + ci * _CHUNK, _CHUNK)])


def _gather_call(table, idx_flat):
    mesh = plsc.VectorSubcoreMesh(core_axis_name="c", subcore_axis_name="s")
    k = pl.kernel(
        _gather_body,
        out_type=jax.ShapeDtypeStruct((_NIDX, 2, C), F32),
        mesh=mesh,
        scratch_types=[
            pltpu.VMEM((_ROWS_W,), jnp.int32),
            pltpu.VMEM((_CHUNK, 2, C), F32),
            pltpu.SemaphoreType.DMA,
        ],
    )
    return k(table, idx_flat)


_TB_E = 256


def _combine_body(q_ref, y_ref, g_ref, wp_ref, gate_ref, out_ref):
    q = q_ref[0].astype(F32)
    y = jnp.concatenate([y_ref[0, h] for h in range(H)], axis=1)

    r16 = lax.broadcasted_iota(jnp.int32, (C, H), 0) // DH
    c16 = lax.broadcasted_iota(jnp.int32, (C, H), 1)
    E16 = (r16 == c16).astype(BF16)
    r64 = lax.broadcasted_iota(jnp.int32, (H, C), 0)
    c64 = lax.broadcasted_iota(jnp.int32, (H, C), 1) // DH
    E64 = (r64 == c64).astype(BF16)

    logits = []
    for kk in range(TOPK):
        gk = g_ref[0, :, kk, 0, :]
        logits.append(jnp.dot((q * gk).astype(BF16), E16,
                              preferred_element_type=F32)
                      * F32(0.125))
    mx = jnp.maximum(jnp.maximum(logits[0], logits[1]), logits[2])
    ws = [jnp.exp(lg - mx) for lg in logits]
    denom = ws[0] + ws[1] + ws[2]
    acc = jnp.zeros((_TB_E, C), F32)
    for kk in range(TOPK):
        gv = g_ref[0, :, kk, 1, :]
        wexp = jnp.dot(ws[kk].astype(BF16), E64, preferred_element_type=F32)
        acc = acc + wexp * gv
    den_exp = jnp.dot(denom.astype(BF16), E64, preferred_element_type=F32)
    mem_qkv = acc / den_exp

    gate = gate_ref[0:1, :]
    combined = mem_qkv * gate + y * (1.0 - gate)
    out_ref[0] = jnp.dot(combined.astype(BF16), wp_ref[...].astype(BF16),
                         preferred_element_type=F32)


def _combine_call(q3, yh, g4, W_proj, gate_row):
    grid = (B, T // _TB_E)
    return pl.pallas_call(
        _combine_body,
        grid=grid,
        in_specs=[
            pl.BlockSpec((1, _TB_E, C), lambda b, tt: (b, tt, 0)),
            pl.BlockSpec((1, H, _TB_E, DH), lambda b, tt: (b, 0, tt, 0)),
            pl.BlockSpec((1, _TB_E, TOPK, 2, C),
                         lambda b, tt: (b, tt, 0, 0, 0)),
            pl.BlockSpec((C, C), lambda b, tt: (0, 0)),
            pl.BlockSpec((8, C), lambda b, tt: (0, 0)),
        ],
        out_specs=pl.BlockSpec((1, _TB_E, C), lambda b, tt: (b, tt, 0)),
        out_shape=jax.ShapeDtypeStruct((B, T, C), F32),
    )(q3, yh, g4, W_proj, gate_row)


def kernel(x, mem_kv, W_attn, W_proj, gate_bias):
    x2 = x.reshape(BT, C)
    q, qh, kh, vh, kvmem = _qkv_call(x2, W_attn)

    q3 = q.reshape(B, T, C)
    yh = _sdpa_call(qh, kh, vh)

    idx8 = _knn_call(q3, mem_kv[:, :, 0, :].astype(BF16))
    idx_flat = idx8[:, :, :TOPK].reshape(_NIDX)

    table = mem_kv.reshape(B * M, 2, C)
    g = _gather_call(table, idx_flat)
    g4 = g.reshape(B, T, TOPK, 2, C)

    gate_vec = jnp.repeat(gate_bias.reshape(H), DH)
    gate_row = jnp.broadcast_to(gate_vec, (8, C))

    out = _combine_call(q3, yh, g4, W_proj, gate_row)
    kv_memories = kvmem.reshape(B, T, 2, C)
    return out, kv_memories
